# Initial kernel scaffold; baseline (speedup 1.0000x reference)
import jax
import jax.numpy as jnp
from jax.experimental import pallas as pl


def kernel(hidden_states, router_W, router_b, W1, b1, W2, b2):
    raise NotImplementedError("write your pallas kernel here")


# speedup vs baseline: 1.2297x; 1.2297x over previous
import functools

import jax
import jax.numpy as jnp
from jax import lax
from jax.experimental import pallas as pl
from jax.experimental.pallas import tpu as pltpu

S, H, E, F = 2048, 768, 8, 3072


def _router_body(x_ref, rw_ref, rb_ref, prob_ref, ind_ref):
    x = x_ref[...]
    rw = rw_ref[...]
    logits = lax.dot_general(x, rw, (((1,), (1,)), ((), ())),
                             preferred_element_type=jnp.float32)
    logits = logits + rb_ref[...]
    m = jnp.max(logits, axis=1, keepdims=True)
    z = jnp.sum(jnp.exp(logits - m), axis=1, keepdims=True)
    prob_ref[...] = 1.0 / z
    iota = lax.broadcasted_iota(jnp.int32, logits.shape, 1)
    ind_ref[...] = jnp.min(jnp.where(logits == m, iota, E), axis=1,
                           keepdims=True)


def _router(x, rw, rb):
    return pl.pallas_call(
        _router_body,
        out_shape=(
            jax.ShapeDtypeStruct((S, 1), jnp.float32),
            jax.ShapeDtypeStruct((S, 1), jnp.int32),
        ),
    )(x, rw, rb.reshape(1, E))


def _gelu(x):
    return 0.5 * x * (1.0 + jnp.tanh(0.7978845608028654 * x
                                     * (1.0 + 0.044715 * x * x)))


def _mlp_body(x_ref, w1_ref, b1_ref, w2_ref, b2_ref, prob_ref, ind_ref,
              out_ref, outb_ref):
    e = pl.program_id(1)
    x = x_ref[...]
    h1 = lax.dot_general(x, w1_ref[0], (((1,), (1,)), ((), ())),
                         preferred_element_type=jnp.float32)
    h1 = _gelu(h1 + b1_ref[0])
    y = lax.dot_general(h1, w2_ref[0], (((1,), (1,)), ((), ())),
                        preferred_element_type=jnp.float32)
    mask = (ind_ref[...] == e).astype(jnp.float32)

    @pl.when(e == 0)
    def _():
        out_ref[...] = jnp.zeros_like(out_ref)
        outb_ref[...] = jnp.zeros_like(outb_ref)

    out_ref[...] += mask * y
    outb_ref[...] += mask * b2_ref[0]

    @pl.when(e == E - 1)
    def _():
        prob = prob_ref[...]
        out_ref[...] *= prob
        outb_ref[...] *= prob


def _mlp(x, w1, b1, w2, b2, prob, ind):
    bt = 512
    nt = S // bt
    return pl.pallas_call(
        _mlp_body,
        grid=(nt, E),
        in_specs=[
            pl.BlockSpec((bt, H), lambda t, e: (t, 0)),
            pl.BlockSpec((1, F, H), lambda t, e: (e, 0, 0)),
            pl.BlockSpec((1, 1, F), lambda t, e: (e, 0, 0)),
            pl.BlockSpec((1, H, F), lambda t, e: (e, 0, 0)),
            pl.BlockSpec((1, 1, H), lambda t, e: (e, 0, 0)),
            pl.BlockSpec((bt, 1), lambda t, e: (t, 0)),
            pl.BlockSpec((bt, 1), lambda t, e: (t, 0)),
        ],
        out_specs=(
            pl.BlockSpec((bt, H), lambda t, e: (t, 0)),
            pl.BlockSpec((bt, H), lambda t, e: (t, 0)),
        ),
        out_shape=(
            jax.ShapeDtypeStruct((S, H), jnp.float32),
            jax.ShapeDtypeStruct((S, H), jnp.float32),
        ),
        compiler_params=pltpu.CompilerParams(
            dimension_semantics=("arbitrary", "arbitrary"),
        ),
    )(x, w1, b1.reshape(E, 1, F), w2, b2.reshape(E, 1, H), prob, ind)


def kernel(hidden_states, router_W, router_b, W1, b1, W2, b2):
    s, b, h = hidden_states.shape
    x = hidden_states.reshape(s * b, h)
    prob, ind = _router(x, router_W, router_b)
    out, outb = _mlp(x, W1, b1, W2, b2, prob, ind)
    return out.reshape(s, b, h), outb.reshape(s, b, h)
